# baseline (speedup 1.0000x reference)
import dataclasses
import functools

import jax
import jax.numpy as jnp
from jax import lax
from jax.experimental import pallas as pl
from jax.experimental.pallas import tpu as pltpu
from jax.experimental.pallas import tpu_sc as plsc

N = 10000
E = 320000
DIN = 128
HID = 128
DOUT = 128
ED = 4
L = 4

NPAD = 10240
NC = 2
NS = 16
NW = NC * NS
GCH = 128
EPAD = 327680
NCHUNK = EPAD // GCH
EPW = EPAD // NW
RPT = NPAD // NS

C_E = 1024
C_N = 1024

_mesh = plsc.VectorSubcoreMesh(core_axis_name="c", subcore_axis_name="s")
F32 = jnp.float32

_cp_no_layout = pltpu.CompilerParams()
if "needs_layout_passes" in pltpu.CompilerParams.__dataclass_fields__:
    _cp_no_layout = dataclasses.replace(_cp_no_layout, needs_layout_passes=False)


def _silu(x):
    return x * jax.nn.sigmoid(x)


def _dot(a, b):
    return jnp.dot(a, b, preferred_element_type=F32)


OCH = 2048


@functools.partial(
    pl.kernel,
    mesh=_mesh,
    out_type=jax.ShapeDtypeStruct((4, EPAD), F32),
    compiler_params=_cp_no_layout,
    scratch_types=[
        pltpu.VMEM((NPAD * 4,), F32),
        pltpu.VMEM((EPW,), jnp.int32),
        pltpu.VMEM((EPW,), jnp.int32),
        pltpu.VMEM((4, OCH), F32),
    ],
)
def _sc_coords(c4f_hbm, colf_hbm, rowf_hbm, out_hbm, ctab, colv, rowv, obuf):
    wid = lax.axis_index("s") * NC + lax.axis_index("c")
    base = wid * EPW
    pltpu.sync_copy(c4f_hbm, ctab)
    pltpu.sync_copy(colf_hbm.at[pl.ds(base, EPW)], colv)
    pltpu.sync_copy(rowf_hbm.at[pl.ds(base, EPW)], rowv)

    for k in range(EPW // OCH):
        @pl.loop(0, OCH // 64)
        def _(i, _k=k):
            for u in range(4):
                off = _k * OCH + i * 64 + u * 16
                pos = i * 64 + u * 16
                c16 = colv[pl.ds(off, 16)] * 4
                r16 = rowv[pl.ds(off, 16)] * 4
                ccx = plsc.load_gather(ctab, [c16])
                ccy = plsc.load_gather(ctab, [c16 + 1])
                ccz = plsc.load_gather(ctab, [c16 + 2])
                crx = plsc.load_gather(ctab, [r16])
                cry = plsc.load_gather(ctab, [r16 + 1])
                crz = plsc.load_gather(ctab, [r16 + 2])
                cdx = crx - ccx
                cdy = cry - ccy
                cdz = crz - ccz
                dist = cdx * cdx + cdy * cdy + cdz * cdz
                obuf[0, pl.ds(pos, 16)] = cdx
                obuf[1, pl.ds(pos, 16)] = cdy
                obuf[2, pl.ds(pos, 16)] = cdz
                obuf[3, pl.ds(pos, 16)] = dist

        pltpu.sync_copy(obuf,
                        out_hbm.at[pl.ds(0, 4), pl.ds(base + k * OCH, OCH)])


@functools.partial(
    pl.kernel,
    mesh=_mesh,
    out_type=(
        jax.ShapeDtypeStruct((EPAD, HID), F32),
        jax.ShapeDtypeStruct((EPAD, HID), F32),
    ),
    scratch_types=[
        pltpu.SemaphoreType.DMA,
        pltpu.SemaphoreType.DMA,
    ],
)
def _sc_gather(ta_hbm, tb_hbm, col_hbm, row_hbm, outc_hbm, outr_hbm,
               semc, semr):
    def body(ic_vmem, ir_vmem, oc_vmem, or_vmem):
        cpc = pltpu.async_copy(ta_hbm.at[ic_vmem.at[0, 0]], oc_vmem, semc)
        cpr = pltpu.async_copy(tb_hbm.at[ir_vmem.at[0, 0]], or_vmem, semr)
        cpc.wait()
        cpr.wait()

    pltpu.emit_pipeline(
        body,
        grid=(NCHUNK,),
        in_specs=[
            pl.BlockSpec((1, 1, GCH), lambda i: (i, 0, 0)),
            pl.BlockSpec((1, 1, GCH), lambda i: (i, 0, 0)),
        ],
        out_specs=[
            pl.BlockSpec((GCH, HID), lambda i: (i, 0)),
            pl.BlockSpec((GCH, HID), lambda i: (i, 0)),
        ],
        core_axis_name=("c", "s"),
        dimension_semantics=(pltpu.PARALLEL,),
    )(col_hbm, row_hbm, outc_hbm, outr_hbm)


@functools.partial(
    pl.kernel,
    mesh=_mesh,
    out_type=jax.ShapeDtypeStruct((NC, 2, NPAD, HID), F32),
    scratch_types=[
        pltpu.VMEM_SHARED((NPAD, HID), F32),
    ],
)
def _sc_scatter(msg_m_hbm, msg_s_hbm, col_hbm, zeros_hbm, out_hbm, acc_sh):
    cid = lax.axis_index("c")
    sid = lax.axis_index("s")

    def _phase(src_hbm, slot):
        pltpu.sync_copy(zeros_hbm.at[pl.ds(sid * RPT, RPT)],
                        acc_sh.at[pl.ds(sid * RPT, RPT)])
        plsc.subcore_barrier()

        def body(m_vmem, i_vmem):
            pltpu.sync_copy(m_vmem, acc_sh.at[i_vmem.at[0, 0]], add=True)

        pltpu.emit_pipeline(
            body,
            grid=(NCHUNK,),
            in_specs=[
                pl.BlockSpec((GCH, HID), lambda i: (i, 0)),
                pl.BlockSpec((1, 1, GCH), lambda i: (i, 0, 0)),
            ],
            out_specs=[],
            core_axis_name=("c", "s"),
            dimension_semantics=(pltpu.PARALLEL,),
        )(src_hbm, col_hbm)

        plsc.subcore_barrier()
        pltpu.sync_copy(acc_sh.at[pl.ds(sid * RPT, RPT)],
                        out_hbm.at[cid, slot, pl.ds(sid * RPT, RPT)])
        plsc.subcore_barrier()

    _phase(msg_m_hbm, 0)
    _phase(msg_s_hbm, 1)


def _embed_body(h_ref, Win_ref, bin_ref, o_ref):
    o_ref[...] = _dot(h_ref[...], Win_ref[...]) + bin_ref[...]


def _project_body(h_ref, Wa, Wb, be1r, ta_ref, tb_ref):
    hv = h_ref[...]
    ta_ref[...] = _dot(hv, Wa[...]) + be1r[...]
    tb_ref[...] = _dot(hv, Wb[...])


def _edge_body(gc_ref, gr_ref, aux_ref, ea_ref, wd, We, We2r, be2r,
               Wc1r, bc1r, Wc2r, m_ref, s_ref):
    aux = lax.dot_general(aux_ref[...], jnp.eye(4, dtype=F32),
                          (((0,), (0,)), ((), ())),
                          preferred_element_type=F32)
    pre = (gc_ref[...] + gr_ref[...] + aux[:, 3:4] * wd[...]
           + _dot(ea_ref[...], We[...]))
    m = _silu(pre)
    m = _silu(_dot(m, We2r[...]) + be2r[...])
    cw = _dot(_silu(_dot(m, Wc1r[...]) + bc1r[...]), Wc2r[...])
    cu = aux[:, 0:3] * cw
    b = m.shape[0]
    m_ref[...] = m
    s_ref[...] = jnp.concatenate(
        [cu, jnp.ones((b, 1), F32), jnp.zeros((b, HID - 4), F32)], axis=1)


def _node_body(h_ref, c4_ref, a0m_ref, a1m_ref, a0s_ref, a1s_ref,
               Wn1a, Wn1b, bn1r, Wn2r, bn2r, h_out_ref, c4_out_ref):
    h = h_ref[...]
    c4 = c4_ref[...]
    agg_feat = a0m_ref[...] + a1m_ref[...]
    s = a0s_ref[...] + a1s_ref[...]
    cnt = jnp.maximum(s[:, 3:4], 1.0)
    agg_coord = s[:, 0:3] / cnt
    u = _silu(_dot(h, Wn1a[...]) + _dot(agg_feat, Wn1b[...]) + bn1r[...])
    upd = _dot(u, Wn2r[...]) + bn2r[...]
    h_out_ref[...] = h + upd
    cn = 2.0 * c4[:, 0:3] + agg_coord
    b = h.shape[0]
    c4_out_ref[...] = jnp.concatenate([cn, jnp.zeros((b, 1), F32)], axis=1)


def _out_body(h_ref, Wout_ref, bout_ref, o_ref):
    o_ref[...] = _dot(h_ref[...], Wout_ref[...]) + bout_ref[...]


def _full(r, c):
    return pl.BlockSpec((r, c), lambda i: (0, 0))


def _tc_embed(h_pad, Win, bin_r):
    return pl.pallas_call(
        _embed_body,
        grid=(NPAD // C_N,),
        in_specs=[
            pl.BlockSpec((C_N, DIN), lambda i: (i, 0)),
            _full(DIN, HID),
            _full(1, HID),
        ],
        out_specs=pl.BlockSpec((C_N, HID), lambda i: (i, 0)),
        out_shape=jax.ShapeDtypeStruct((NPAD, HID), F32),
    )(h_pad, Win, bin_r)


def _tc_project(hs, Wa, Wb, be1r):
    return pl.pallas_call(
        _project_body,
        grid=(NPAD // C_N,),
        in_specs=[
            pl.BlockSpec((C_N, HID), lambda i: (i, 0)),
            _full(HID, HID),
            _full(HID, HID),
            _full(1, HID),
        ],
        out_specs=[
            pl.BlockSpec((C_N, HID), lambda i: (i, 0)),
            pl.BlockSpec((C_N, HID), lambda i: (i, 0)),
        ],
        out_shape=[
            jax.ShapeDtypeStruct((NPAD, HID), F32),
            jax.ShapeDtypeStruct((NPAD, HID), F32),
        ],
    )(hs, Wa, Wb, be1r)


def _tc_edge(gc, gr, aux, ea_pad, wd, We, We2r, be2r, Wc1r, bc1r, Wc2r):
    return pl.pallas_call(
        _edge_body,
        grid=(EPAD // C_E,),
        in_specs=[
            pl.BlockSpec((C_E, HID), lambda i: (i, 0)),
            pl.BlockSpec((C_E, HID), lambda i: (i, 0)),
            pl.BlockSpec((4, C_E), lambda i: (0, i)),
            pl.BlockSpec((C_E, ED), lambda i: (i, 0)),
            _full(1, HID),
            _full(ED, HID),
            _full(HID, HID),
            _full(1, HID),
            _full(HID, HID),
            _full(1, HID),
            _full(HID, 1),
        ],
        out_specs=[
            pl.BlockSpec((C_E, HID), lambda i: (i, 0)),
            pl.BlockSpec((C_E, HID), lambda i: (i, 0)),
        ],
        out_shape=[
            jax.ShapeDtypeStruct((EPAD, HID), F32),
            jax.ShapeDtypeStruct((EPAD, HID), F32),
        ],
    )(gc, gr, aux, ea_pad, wd, We, We2r, be2r, Wc1r, bc1r, Wc2r)


def _tc_node(hs, c4, a0m, a1m, a0s, a1s, Wn1a, Wn1b, bn1r, Wn2r, bn2r):
    return pl.pallas_call(
        _node_body,
        grid=(NPAD // C_N,),
        in_specs=[
            pl.BlockSpec((C_N, HID), lambda i: (i, 0)),
            pl.BlockSpec((C_N, 4), lambda i: (i, 0)),
            pl.BlockSpec((C_N, HID), lambda i: (i, 0)),
            pl.BlockSpec((C_N, HID), lambda i: (i, 0)),
            pl.BlockSpec((C_N, HID), lambda i: (i, 0)),
            pl.BlockSpec((C_N, HID), lambda i: (i, 0)),
            _full(HID, HID),
            _full(HID, HID),
            _full(1, HID),
            _full(HID, HID),
            _full(1, HID),
        ],
        out_specs=[
            pl.BlockSpec((C_N, HID), lambda i: (i, 0)),
            pl.BlockSpec((C_N, 4), lambda i: (i, 0)),
        ],
        out_shape=[
            jax.ShapeDtypeStruct((NPAD, HID), F32),
            jax.ShapeDtypeStruct((NPAD, 4), F32),
        ],
    )(hs, c4, a0m, a1m, a0s, a1s, Wn1a, Wn1b, bn1r, Wn2r, bn2r)


def _tc_out(hs, Wout, bout_r):
    return pl.pallas_call(
        _out_body,
        grid=(NPAD // C_N,),
        in_specs=[
            pl.BlockSpec((C_N, HID), lambda i: (i, 0)),
            _full(HID, DOUT),
            _full(1, DOUT),
        ],
        out_specs=pl.BlockSpec((C_N, DOUT), lambda i: (i, 0)),
        out_shape=jax.ShapeDtypeStruct((NPAD, DOUT), F32),
    )(hs, Wout, bout_r)


def kernel(h, coords, edge_index, edge_attr, Win, bin_, Wout, bout,
           We1, be1, We2, be2, Wn1, bn1, Wn2, bn2, Wc1, bc1, Wc2):
    row = edge_index[0].astype(jnp.int32)
    col = edge_index[1].astype(jnp.int32)
    pad_e = EPAD - E
    col_flat = jnp.concatenate([col, jnp.full((pad_e,), NPAD - 1, jnp.int32)])
    row_flat = jnp.concatenate([row, jnp.zeros((pad_e,), jnp.int32)])
    col_pad = col_flat.reshape(NCHUNK, 1, GCH)
    row_pad = row_flat.reshape(NCHUNK, 1, GCH)
    ea_pad = jnp.concatenate(
        [edge_attr, jnp.zeros((pad_e, ED), F32)], axis=0)
    h_pad = jnp.concatenate([h, jnp.zeros((NPAD - N, DIN), F32)], axis=0)
    c4 = jnp.concatenate(
        [jnp.concatenate([coords, jnp.zeros((N, 1), F32)], axis=1),
         jnp.zeros((NPAD - N, 4), F32)], axis=0)

    zeros128 = jnp.zeros((NPAD, HID), F32)

    hs = _tc_embed(h_pad, Win, bin_.reshape(1, HID))

    for l in range(L):
        Wa = We1[l, 0:HID]
        Wb = We1[l, HID:2 * HID]
        wd = We1[l, 2 * HID:2 * HID + 1]
        We = We1[l, 2 * HID + 1:]
        ta, tb = _tc_project(hs, Wa, Wb, be1[l].reshape(1, HID))
        aux = _sc_coords(c4.reshape(NPAD * 4), col_flat, row_flat)
        gc, gr = _sc_gather(ta, tb, col_pad, row_pad)
        msg_m, msg_s = _tc_edge(gc, gr, aux, ea_pad, wd, We, We2[l],
                                be2[l].reshape(1, HID), Wc1[l],
                                bc1[l].reshape(1, HID), Wc2[l])
        agg = _sc_scatter(msg_m, msg_s, col_pad, zeros128)
        hs, c4 = _tc_node(hs, c4, agg[0, 0], agg[1, 0], agg[0, 1], agg[1, 1],
                          Wn1[l, :HID], Wn1[l, HID:], bn1[l].reshape(1, HID),
                          Wn2[l], bn2[l].reshape(1, HID))

    h_out = _tc_out(hs, Wout, bout.reshape(1, DOUT))
    return (h_out[:N], c4[:N, 0:3])


# speedup vs baseline: 1.6232x; 1.6232x over previous
import dataclasses
import functools

import jax
import jax.numpy as jnp
from jax import lax
from jax.experimental import pallas as pl
from jax.experimental.pallas import tpu as pltpu
from jax.experimental.pallas import tpu_sc as plsc

N = 10000
E = 320000
DIN = 128
HID = 128
DOUT = 128
ED = 4
L = 4

NPAD = 10240
NC = 2
NS = 16
NW = NC * NS
GCH = 128
EPAD = 327680
NCHUNK = EPAD // GCH
EPW = EPAD // NW
RPT = NPAD // NS

C_E = 1024
C_N = 1024

_mesh = plsc.VectorSubcoreMesh(core_axis_name="c", subcore_axis_name="s")
F32 = jnp.float32

_cp_no_layout = pltpu.CompilerParams()
if "needs_layout_passes" in pltpu.CompilerParams.__dataclass_fields__:
    _cp_no_layout = dataclasses.replace(_cp_no_layout, needs_layout_passes=False)


def _silu(x):
    return x * jax.nn.sigmoid(x)


def _dot(a, b):
    return jnp.dot(a, b, preferred_element_type=F32)


OCH = 2048


@functools.partial(
    pl.kernel,
    mesh=_mesh,
    out_type=jax.ShapeDtypeStruct((4, EPAD), F32),
    compiler_params=_cp_no_layout,
    scratch_types=[
        pltpu.VMEM((NPAD * 4,), F32),
        pltpu.VMEM((EPW,), jnp.int32),
        pltpu.VMEM((EPW,), jnp.int32),
        pltpu.VMEM((4, OCH), F32),
    ],
)
def _sc_coords(c4f_hbm, colf_hbm, rowf_hbm, out_hbm, ctab, colv, rowv, obuf):
    wid = lax.axis_index("s") * NC + lax.axis_index("c")
    base = wid * EPW
    pltpu.sync_copy(c4f_hbm, ctab)
    pltpu.sync_copy(colf_hbm.at[pl.ds(base, EPW)], colv)
    pltpu.sync_copy(rowf_hbm.at[pl.ds(base, EPW)], rowv)

    for k in range(EPW // OCH):
        @pl.loop(0, OCH // 64)
        def _(i, _k=k):
            for u in range(4):
                off = _k * OCH + i * 64 + u * 16
                pos = i * 64 + u * 16
                c16 = colv[pl.ds(off, 16)] * 4
                r16 = rowv[pl.ds(off, 16)] * 4
                ccx = plsc.load_gather(ctab, [c16])
                ccy = plsc.load_gather(ctab, [c16 + 1])
                ccz = plsc.load_gather(ctab, [c16 + 2])
                crx = plsc.load_gather(ctab, [r16])
                cry = plsc.load_gather(ctab, [r16 + 1])
                crz = plsc.load_gather(ctab, [r16 + 2])
                cdx = crx - ccx
                cdy = cry - ccy
                cdz = crz - ccz
                dist = cdx * cdx + cdy * cdy + cdz * cdz
                obuf[0, pl.ds(pos, 16)] = cdx
                obuf[1, pl.ds(pos, 16)] = cdy
                obuf[2, pl.ds(pos, 16)] = cdz
                obuf[3, pl.ds(pos, 16)] = dist

        pltpu.sync_copy(obuf,
                        out_hbm.at[pl.ds(0, 4), pl.ds(base + k * OCH, OCH)])


@functools.partial(
    pl.kernel,
    mesh=_mesh,
    out_type=(
        jax.ShapeDtypeStruct((EPAD, HID), F32),
        jax.ShapeDtypeStruct((EPAD, HID), F32),
    ),
    scratch_types=[
        pltpu.VMEM_SHARED((NPAD, HID), F32),
        pltpu.SemaphoreType.DMA,
    ],
)
def _sc_gather(ta_hbm, tb_hbm, col_hbm, row_hbm, outc_hbm, outr_hbm,
               tbl_sh, sem):
    cid = lax.axis_index("c")
    sid = lax.axis_index("s")

    @pl.when(cid == 0)
    def _():
        pltpu.sync_copy(ta_hbm.at[pl.ds(sid * RPT, RPT)],
                        tbl_sh.at[pl.ds(sid * RPT, RPT)])

    @pl.when(cid == 1)
    def _():
        pltpu.sync_copy(tb_hbm.at[pl.ds(sid * RPT, RPT)],
                        tbl_sh.at[pl.ds(sid * RPT, RPT)])

    plsc.subcore_barrier()

    def body(i_vmem, o_vmem):
        pltpu.async_copy(tbl_sh.at[i_vmem.at[0, 0]], o_vmem, sem).wait()

    def _run(idx_hbm, out_hbm):
        pltpu.emit_pipeline(
            body,
            grid=(NCHUNK,),
            in_specs=[pl.BlockSpec((1, 1, GCH), lambda i: (i, 0, 0))],
            out_specs=[pl.BlockSpec((GCH, HID), lambda i: (i, 0))],
            core_axis_name="s",
            dimension_semantics=(pltpu.PARALLEL,),
        )(idx_hbm, out_hbm)

    @pl.when(cid == 0)
    def _():
        _run(col_hbm, outc_hbm)

    @pl.when(cid == 1)
    def _():
        _run(row_hbm, outr_hbm)


@functools.partial(
    pl.kernel,
    mesh=_mesh,
    out_type=jax.ShapeDtypeStruct((2, NPAD, HID), F32),
    scratch_types=[
        pltpu.VMEM_SHARED((NPAD, HID), F32),
    ],
)
def _sc_scatter(msg_m_hbm, msg_s_hbm, col_hbm, zeros_hbm, out_hbm, acc_sh):
    cid = lax.axis_index("c")
    sid = lax.axis_index("s")

    pltpu.sync_copy(zeros_hbm.at[pl.ds(sid * RPT, RPT)],
                    acc_sh.at[pl.ds(sid * RPT, RPT)])
    plsc.subcore_barrier()

    def body(m_vmem, i_vmem):
        pltpu.sync_copy(m_vmem, acc_sh.at[i_vmem.at[0, 0]], add=True)

    def _run(src_hbm):
        pltpu.emit_pipeline(
            body,
            grid=(NCHUNK,),
            in_specs=[
                pl.BlockSpec((GCH, HID), lambda i: (i, 0)),
                pl.BlockSpec((1, 1, GCH), lambda i: (i, 0, 0)),
            ],
            out_specs=[],
            core_axis_name="s",
            dimension_semantics=(pltpu.PARALLEL,),
        )(src_hbm, col_hbm)

    @pl.when(cid == 0)
    def _():
        _run(msg_m_hbm)

    @pl.when(cid == 1)
    def _():
        _run(msg_s_hbm)

    plsc.subcore_barrier()
    pltpu.sync_copy(acc_sh.at[pl.ds(sid * RPT, RPT)],
                    out_hbm.at[cid, pl.ds(sid * RPT, RPT)])


def _embed_body(h_ref, Win_ref, bin_ref, o_ref):
    o_ref[...] = _dot(h_ref[...], Win_ref[...]) + bin_ref[...]


def _project_body(h_ref, Wa, Wb, be1r, ta_ref, tb_ref):
    hv = h_ref[...]
    ta_ref[...] = _dot(hv, Wa[...]) + be1r[...]
    tb_ref[...] = _dot(hv, Wb[...])


def _edge_body(gc_ref, gr_ref, aux_ref, ea_ref, wd, We, We2r, be2r,
               Wc1r, bc1r, Wc2r, m_ref, s_ref):
    aux = lax.dot_general(aux_ref[...], jnp.eye(4, dtype=F32),
                          (((0,), (0,)), ((), ())),
                          preferred_element_type=F32)
    pre = (gc_ref[...] + gr_ref[...] + aux[:, 3:4] * wd[...]
           + _dot(ea_ref[...], We[...]))
    m = _silu(pre)
    m = _silu(_dot(m, We2r[...]) + be2r[...])
    cw = _dot(_silu(_dot(m, Wc1r[...]) + bc1r[...]), Wc2r[...])
    cu = aux[:, 0:3] * cw
    b = m.shape[0]
    m_ref[...] = m
    s_ref[...] = jnp.concatenate(
        [cu, jnp.ones((b, 1), F32), jnp.zeros((b, HID - 4), F32)], axis=1)


def _node_body(h_ref, c4_ref, am_ref, as_ref,
               Wn1a, Wn1b, bn1r, Wn2r, bn2r, h_out_ref, c4_out_ref):
    h = h_ref[...]
    c4 = c4_ref[...]
    agg_feat = am_ref[...]
    s = as_ref[...]
    cnt = jnp.maximum(s[:, 3:4], 1.0)
    agg_coord = s[:, 0:3] / cnt
    u = _silu(_dot(h, Wn1a[...]) + _dot(agg_feat, Wn1b[...]) + bn1r[...])
    upd = _dot(u, Wn2r[...]) + bn2r[...]
    h_out_ref[...] = h + upd
    cn = 2.0 * c4[:, 0:3] + agg_coord
    b = h.shape[0]
    c4_out_ref[...] = jnp.concatenate([cn, jnp.zeros((b, 1), F32)], axis=1)


def _out_body(h_ref, Wout_ref, bout_ref, o_ref):
    o_ref[...] = _dot(h_ref[...], Wout_ref[...]) + bout_ref[...]


def _full(r, c):
    return pl.BlockSpec((r, c), lambda i: (0, 0))


def _tc_embed(h_pad, Win, bin_r):
    return pl.pallas_call(
        _embed_body,
        grid=(NPAD // C_N,),
        in_specs=[
            pl.BlockSpec((C_N, DIN), lambda i: (i, 0)),
            _full(DIN, HID),
            _full(1, HID),
        ],
        out_specs=pl.BlockSpec((C_N, HID), lambda i: (i, 0)),
        out_shape=jax.ShapeDtypeStruct((NPAD, HID), F32),
    )(h_pad, Win, bin_r)


def _tc_project(hs, Wa, Wb, be1r):
    return pl.pallas_call(
        _project_body,
        grid=(NPAD // C_N,),
        in_specs=[
            pl.BlockSpec((C_N, HID), lambda i: (i, 0)),
            _full(HID, HID),
            _full(HID, HID),
            _full(1, HID),
        ],
        out_specs=[
            pl.BlockSpec((C_N, HID), lambda i: (i, 0)),
            pl.BlockSpec((C_N, HID), lambda i: (i, 0)),
        ],
        out_shape=[
            jax.ShapeDtypeStruct((NPAD, HID), F32),
            jax.ShapeDtypeStruct((NPAD, HID), F32),
        ],
    )(hs, Wa, Wb, be1r)


def _tc_edge(gc, gr, aux, ea_pad, wd, We, We2r, be2r, Wc1r, bc1r, Wc2r):
    return pl.pallas_call(
        _edge_body,
        grid=(EPAD // C_E,),
        in_specs=[
            pl.BlockSpec((C_E, HID), lambda i: (i, 0)),
            pl.BlockSpec((C_E, HID), lambda i: (i, 0)),
            pl.BlockSpec((4, C_E), lambda i: (0, i)),
            pl.BlockSpec((C_E, ED), lambda i: (i, 0)),
            _full(1, HID),
            _full(ED, HID),
            _full(HID, HID),
            _full(1, HID),
            _full(HID, HID),
            _full(1, HID),
            _full(HID, 1),
        ],
        out_specs=[
            pl.BlockSpec((C_E, HID), lambda i: (i, 0)),
            pl.BlockSpec((C_E, HID), lambda i: (i, 0)),
        ],
        out_shape=[
            jax.ShapeDtypeStruct((EPAD, HID), F32),
            jax.ShapeDtypeStruct((EPAD, HID), F32),
        ],
    )(gc, gr, aux, ea_pad, wd, We, We2r, be2r, Wc1r, bc1r, Wc2r)


def _tc_node(hs, c4, am, as_, Wn1a, Wn1b, bn1r, Wn2r, bn2r):
    return pl.pallas_call(
        _node_body,
        grid=(NPAD // C_N,),
        in_specs=[
            pl.BlockSpec((C_N, HID), lambda i: (i, 0)),
            pl.BlockSpec((C_N, 4), lambda i: (i, 0)),
            pl.BlockSpec((C_N, HID), lambda i: (i, 0)),
            pl.BlockSpec((C_N, HID), lambda i: (i, 0)),
            _full(HID, HID),
            _full(HID, HID),
            _full(1, HID),
            _full(HID, HID),
            _full(1, HID),
        ],
        out_specs=[
            pl.BlockSpec((C_N, HID), lambda i: (i, 0)),
            pl.BlockSpec((C_N, 4), lambda i: (i, 0)),
        ],
        out_shape=[
            jax.ShapeDtypeStruct((NPAD, HID), F32),
            jax.ShapeDtypeStruct((NPAD, 4), F32),
        ],
    )(hs, c4, am, as_, Wn1a, Wn1b, bn1r, Wn2r, bn2r)


def _tc_out(hs, Wout, bout_r):
    return pl.pallas_call(
        _out_body,
        grid=(NPAD // C_N,),
        in_specs=[
            pl.BlockSpec((C_N, HID), lambda i: (i, 0)),
            _full(HID, DOUT),
            _full(1, DOUT),
        ],
        out_specs=pl.BlockSpec((C_N, DOUT), lambda i: (i, 0)),
        out_shape=jax.ShapeDtypeStruct((NPAD, DOUT), F32),
    )(hs, Wout, bout_r)


def kernel(h, coords, edge_index, edge_attr, Win, bin_, Wout, bout,
           We1, be1, We2, be2, Wn1, bn1, Wn2, bn2, Wc1, bc1, Wc2):
    row = edge_index[0].astype(jnp.int32)
    col = edge_index[1].astype(jnp.int32)
    pad_e = EPAD - E
    col_flat = jnp.concatenate([col, jnp.full((pad_e,), NPAD - 1, jnp.int32)])
    row_flat = jnp.concatenate([row, jnp.zeros((pad_e,), jnp.int32)])
    col_pad = col_flat.reshape(NCHUNK, 1, GCH)
    row_pad = row_flat.reshape(NCHUNK, 1, GCH)
    ea_pad = jnp.concatenate(
        [edge_attr, jnp.zeros((pad_e, ED), F32)], axis=0)
    h_pad = jnp.concatenate([h, jnp.zeros((NPAD - N, DIN), F32)], axis=0)
    c4 = jnp.concatenate(
        [jnp.concatenate([coords, jnp.zeros((N, 1), F32)], axis=1),
         jnp.zeros((NPAD - N, 4), F32)], axis=0)

    zeros128 = jnp.zeros((NPAD, HID), F32)

    hs = _tc_embed(h_pad, Win, bin_.reshape(1, HID))

    for l in range(L):
        Wa = We1[l, 0:HID]
        Wb = We1[l, HID:2 * HID]
        wd = We1[l, 2 * HID:2 * HID + 1]
        We = We1[l, 2 * HID + 1:]
        ta, tb = _tc_project(hs, Wa, Wb, be1[l].reshape(1, HID))
        aux = _sc_coords(c4.reshape(NPAD * 4), col_flat, row_flat)
        gc, gr = _sc_gather(ta, tb, col_pad, row_pad)
        msg_m, msg_s = _tc_edge(gc, gr, aux, ea_pad, wd, We, We2[l],
                                be2[l].reshape(1, HID), Wc1[l],
                                bc1[l].reshape(1, HID), Wc2[l])
        agg = _sc_scatter(msg_m, msg_s, col_pad, zeros128)
        hs, c4 = _tc_node(hs, c4, agg[0], agg[1],
                          Wn1[l, :HID], Wn1[l, HID:], bn1[l].reshape(1, HID),
                          Wn2[l], bn2[l].reshape(1, HID))

    h_out = _tc_out(hs, Wout, bout.reshape(1, DOUT))
    return (h_out[:N], c4[:N, 0:3])


# speedup vs baseline: 1.6465x; 1.0144x over previous
import dataclasses
import functools

import jax
import jax.numpy as jnp
from jax import lax
from jax.experimental import pallas as pl
from jax.experimental.pallas import tpu as pltpu
from jax.experimental.pallas import tpu_sc as plsc

N = 10000
E = 320000
DIN = 128
HID = 128
DOUT = 128
ED = 4
L = 4

NPAD = 10240
NC = 2
NS = 16
NW = NC * NS
GCH = 128
EPAD = 327680
NCHUNK = EPAD // GCH
EPW = EPAD // NW
RPT = NPAD // NS

C_E = 1024
C_N = 1024

_mesh = plsc.VectorSubcoreMesh(core_axis_name="c", subcore_axis_name="s")
F32 = jnp.float32

_cp_no_layout = pltpu.CompilerParams()
if "needs_layout_passes" in pltpu.CompilerParams.__dataclass_fields__:
    _cp_no_layout = dataclasses.replace(_cp_no_layout, needs_layout_passes=False)


def _silu(x):
    return x * jax.nn.sigmoid(x)


def _dot(a, b):
    return jnp.dot(a, b, preferred_element_type=F32)


OCH = 2048


@functools.partial(
    pl.kernel,
    mesh=_mesh,
    out_type=jax.ShapeDtypeStruct((4, EPAD), F32),
    compiler_params=_cp_no_layout,
    scratch_types=[
        pltpu.VMEM((NPAD * 4,), F32),
        pltpu.VMEM((EPW,), jnp.int32),
        pltpu.VMEM((EPW,), jnp.int32),
        pltpu.VMEM((4, OCH), F32),
    ],
)
def _sc_coords(c4f_hbm, colf_hbm, rowf_hbm, out_hbm, ctab, colv, rowv, obuf):
    wid = lax.axis_index("s") * NC + lax.axis_index("c")
    base = wid * EPW
    pltpu.sync_copy(c4f_hbm, ctab)
    pltpu.sync_copy(colf_hbm.at[pl.ds(base, EPW)], colv)
    pltpu.sync_copy(rowf_hbm.at[pl.ds(base, EPW)], rowv)

    for k in range(EPW // OCH):
        @pl.loop(0, OCH // 64)
        def _(i, _k=k):
            for u in range(4):
                off = _k * OCH + i * 64 + u * 16
                pos = i * 64 + u * 16
                c16 = colv[pl.ds(off, 16)] * 4
                r16 = rowv[pl.ds(off, 16)] * 4
                ccx = plsc.load_gather(ctab, [c16])
                ccy = plsc.load_gather(ctab, [c16 + 1])
                ccz = plsc.load_gather(ctab, [c16 + 2])
                crx = plsc.load_gather(ctab, [r16])
                cry = plsc.load_gather(ctab, [r16 + 1])
                crz = plsc.load_gather(ctab, [r16 + 2])
                cdx = crx - ccx
                cdy = cry - ccy
                cdz = crz - ccz
                dist = cdx * cdx + cdy * cdy + cdz * cdz
                obuf[0, pl.ds(pos, 16)] = cdx
                obuf[1, pl.ds(pos, 16)] = cdy
                obuf[2, pl.ds(pos, 16)] = cdz
                obuf[3, pl.ds(pos, 16)] = dist

        pltpu.sync_copy(obuf,
                        out_hbm.at[pl.ds(0, 4), pl.ds(base + k * OCH, OCH)])


@functools.partial(
    pl.kernel,
    mesh=_mesh,
    out_type=(
        jax.ShapeDtypeStruct((EPAD, HID), F32),
        jax.ShapeDtypeStruct((EPAD, HID), F32),
    ),
    scratch_types=[
        pltpu.VMEM_SHARED((NPAD, HID), F32),
        pltpu.SemaphoreType.DMA,
    ],
)
def _sc_gather(ta_hbm, tb_hbm, col_hbm, row_hbm, outc_hbm, outr_hbm,
               tbl_sh, sem):
    cid = lax.axis_index("c")
    sid = lax.axis_index("s")

    @pl.when(cid == 0)
    def _():
        pltpu.sync_copy(ta_hbm.at[pl.ds(sid * RPT, RPT)],
                        tbl_sh.at[pl.ds(sid * RPT, RPT)])

    @pl.when(cid == 1)
    def _():
        pltpu.sync_copy(tb_hbm.at[pl.ds(sid * RPT, RPT)],
                        tbl_sh.at[pl.ds(sid * RPT, RPT)])

    plsc.subcore_barrier()

    def body(i_vmem, o_vmem):
        pltpu.async_copy(tbl_sh.at[i_vmem.at[0, 0]], o_vmem, sem).wait()

    def _run(idx_hbm, out_hbm):
        pltpu.emit_pipeline(
            body,
            grid=(NCHUNK,),
            in_specs=[pl.BlockSpec((1, 1, GCH), lambda i: (i, 0, 0))],
            out_specs=[pl.BlockSpec((GCH, HID), lambda i: (i, 0))],
            core_axis_name="s",
            dimension_semantics=(pltpu.PARALLEL,),
        )(idx_hbm, out_hbm)

    @pl.when(cid == 0)
    def _():
        _run(col_hbm, outc_hbm)

    @pl.when(cid == 1)
    def _():
        _run(row_hbm, outr_hbm)


@functools.partial(
    pl.kernel,
    mesh=_mesh,
    out_type=jax.ShapeDtypeStruct((2, NPAD, HID), F32),
    scratch_types=[
        pltpu.VMEM_SHARED((NPAD, HID), F32),
    ],
)
def _sc_scatter(msg_m_hbm, msg_s_hbm, col_hbm, zeros_hbm, out_hbm, acc_sh):
    cid = lax.axis_index("c")
    sid = lax.axis_index("s")

    pltpu.sync_copy(zeros_hbm.at[pl.ds(sid * RPT, RPT)],
                    acc_sh.at[pl.ds(sid * RPT, RPT)])
    plsc.subcore_barrier()

    def body(m_vmem, i_vmem):
        pltpu.sync_copy(m_vmem, acc_sh.at[i_vmem.at[0, 0]], add=True)

    def _run(src_hbm):
        pltpu.emit_pipeline(
            body,
            grid=(NCHUNK,),
            in_specs=[
                pl.BlockSpec((GCH, HID), lambda i: (i, 0)),
                pl.BlockSpec((1, 1, GCH), lambda i: (i, 0, 0)),
            ],
            out_specs=[],
            core_axis_name="s",
            dimension_semantics=(pltpu.PARALLEL,),
        )(src_hbm, col_hbm)

    @pl.when(cid == 0)
    def _():
        _run(msg_m_hbm)

    @pl.when(cid == 1)
    def _():
        _run(msg_s_hbm)

    plsc.subcore_barrier()
    pltpu.sync_copy(acc_sh.at[pl.ds(sid * RPT, RPT)],
                    out_hbm.at[cid, pl.ds(sid * RPT, RPT)])


def _embed_body(h_ref, Win_ref, bin_ref, Wa, Wb, be1r, o_ref, ta_ref, tb_ref):
    h0 = _dot(h_ref[...], Win_ref[...]) + bin_ref[...]
    o_ref[...] = h0
    ta_ref[...] = _dot(h0, Wa[...]) + be1r[...]
    tb_ref[...] = _dot(h0, Wb[...])


def _edge_body(gc_ref, gr_ref, aux_ref, ea_ref, wd, We, We2r, be2r,
               Wc1r, bc1r, Wc2r, m_ref, s_ref):
    aux = lax.dot_general(aux_ref[...], jnp.eye(4, dtype=F32),
                          (((0,), (0,)), ((), ())),
                          preferred_element_type=F32)
    pre = (gc_ref[...] + gr_ref[...] + aux[:, 3:4] * wd[...]
           + _dot(ea_ref[...], We[...]))
    m = _silu(pre)
    m = _silu(_dot(m, We2r[...]) + be2r[...])
    cw = _dot(_silu(_dot(m, Wc1r[...]) + bc1r[...]), Wc2r[...])
    cu = aux[:, 0:3] * cw
    b = m.shape[0]
    m_ref[...] = m
    s_ref[...] = jnp.concatenate(
        [cu, jnp.ones((b, 1), F32), jnp.zeros((b, HID - 4), F32)], axis=1)


def _node_core(h_ref, c4_ref, am_ref, as_ref, Wn1a, Wn1b, bn1r, Wn2r, bn2r):
    h = h_ref[...]
    c4 = c4_ref[...]
    agg_feat = am_ref[...]
    s = as_ref[...]
    cnt = jnp.maximum(s[:, 3:4], 1.0)
    agg_coord = s[:, 0:3] / cnt
    u = _silu(_dot(h, Wn1a[...]) + _dot(agg_feat, Wn1b[...]) + bn1r[...])
    upd = _dot(u, Wn2r[...]) + bn2r[...]
    hn = h + upd
    cn = 2.0 * c4[:, 0:3] + agg_coord
    b = h.shape[0]
    c4n = jnp.concatenate([cn, jnp.zeros((b, 1), F32)], axis=1)
    return hn, c4n


def _node_body(h_ref, c4_ref, am_ref, as_ref,
               Wn1a, Wn1b, bn1r, Wn2r, bn2r, h_out_ref, c4_out_ref):
    hn, c4n = _node_core(h_ref, c4_ref, am_ref, as_ref,
                         Wn1a, Wn1b, bn1r, Wn2r, bn2r)
    h_out_ref[...] = hn
    c4_out_ref[...] = c4n


def _node_proj_body(h_ref, c4_ref, am_ref, as_ref,
                    Wn1a, Wn1b, bn1r, Wn2r, bn2r, Wa, Wb, be1r,
                    h_out_ref, c4_out_ref, ta_ref, tb_ref):
    hn, c4n = _node_core(h_ref, c4_ref, am_ref, as_ref,
                         Wn1a, Wn1b, bn1r, Wn2r, bn2r)
    h_out_ref[...] = hn
    c4_out_ref[...] = c4n
    ta_ref[...] = _dot(hn, Wa[...]) + be1r[...]
    tb_ref[...] = _dot(hn, Wb[...])


def _out_body(h_ref, Wout_ref, bout_ref, o_ref):
    o_ref[...] = _dot(h_ref[...], Wout_ref[...]) + bout_ref[...]


def _full(r, c):
    return pl.BlockSpec((r, c), lambda i: (0, 0))


def _tc_embed(h_pad, Win, bin_r, Wa, Wb, be1r):
    return pl.pallas_call(
        _embed_body,
        grid=(NPAD // C_N,),
        in_specs=[
            pl.BlockSpec((C_N, DIN), lambda i: (i, 0)),
            _full(DIN, HID),
            _full(1, HID),
            _full(HID, HID),
            _full(HID, HID),
            _full(1, HID),
        ],
        out_specs=[
            pl.BlockSpec((C_N, HID), lambda i: (i, 0)),
            pl.BlockSpec((C_N, HID), lambda i: (i, 0)),
            pl.BlockSpec((C_N, HID), lambda i: (i, 0)),
        ],
        out_shape=[
            jax.ShapeDtypeStruct((NPAD, HID), F32),
            jax.ShapeDtypeStruct((NPAD, HID), F32),
            jax.ShapeDtypeStruct((NPAD, HID), F32),
        ],
    )(h_pad, Win, bin_r, Wa, Wb, be1r)


def _tc_edge(gc, gr, aux, ea_pad, wd, We, We2r, be2r, Wc1r, bc1r, Wc2r):
    return pl.pallas_call(
        _edge_body,
        grid=(EPAD // C_E,),
        in_specs=[
            pl.BlockSpec((C_E, HID), lambda i: (i, 0)),
            pl.BlockSpec((C_E, HID), lambda i: (i, 0)),
            pl.BlockSpec((4, C_E), lambda i: (0, i)),
            pl.BlockSpec((C_E, ED), lambda i: (i, 0)),
            _full(1, HID),
            _full(ED, HID),
            _full(HID, HID),
            _full(1, HID),
            _full(HID, HID),
            _full(1, HID),
            _full(HID, 1),
        ],
        out_specs=[
            pl.BlockSpec((C_E, HID), lambda i: (i, 0)),
            pl.BlockSpec((C_E, HID), lambda i: (i, 0)),
        ],
        out_shape=[
            jax.ShapeDtypeStruct((EPAD, HID), F32),
            jax.ShapeDtypeStruct((EPAD, HID), F32),
        ],
    )(gc, gr, aux, ea_pad, wd, We, We2r, be2r, Wc1r, bc1r, Wc2r)


def _tc_node(hs, c4, am, as_, Wn1a, Wn1b, bn1r, Wn2r, bn2r):
    return pl.pallas_call(
        _node_body,
        grid=(NPAD // C_N,),
        in_specs=[
            pl.BlockSpec((C_N, HID), lambda i: (i, 0)),
            pl.BlockSpec((C_N, 4), lambda i: (i, 0)),
            pl.BlockSpec((C_N, HID), lambda i: (i, 0)),
            pl.BlockSpec((C_N, HID), lambda i: (i, 0)),
            _full(HID, HID),
            _full(HID, HID),
            _full(1, HID),
            _full(HID, HID),
            _full(1, HID),
        ],
        out_specs=[
            pl.BlockSpec((C_N, HID), lambda i: (i, 0)),
            pl.BlockSpec((C_N, 4), lambda i: (i, 0)),
        ],
        out_shape=[
            jax.ShapeDtypeStruct((NPAD, HID), F32),
            jax.ShapeDtypeStruct((NPAD, 4), F32),
        ],
    )(hs, c4, am, as_, Wn1a, Wn1b, bn1r, Wn2r, bn2r)


def _tc_node_proj(hs, c4, am, as_, Wn1a, Wn1b, bn1r, Wn2r, bn2r,
                  Wa, Wb, be1r):
    return pl.pallas_call(
        _node_proj_body,
        grid=(NPAD // C_N,),
        in_specs=[
            pl.BlockSpec((C_N, HID), lambda i: (i, 0)),
            pl.BlockSpec((C_N, 4), lambda i: (i, 0)),
            pl.BlockSpec((C_N, HID), lambda i: (i, 0)),
            pl.BlockSpec((C_N, HID), lambda i: (i, 0)),
            _full(HID, HID),
            _full(HID, HID),
            _full(1, HID),
            _full(HID, HID),
            _full(1, HID),
            _full(HID, HID),
            _full(HID, HID),
            _full(1, HID),
        ],
        out_specs=[
            pl.BlockSpec((C_N, HID), lambda i: (i, 0)),
            pl.BlockSpec((C_N, 4), lambda i: (i, 0)),
            pl.BlockSpec((C_N, HID), lambda i: (i, 0)),
            pl.BlockSpec((C_N, HID), lambda i: (i, 0)),
        ],
        out_shape=[
            jax.ShapeDtypeStruct((NPAD, HID), F32),
            jax.ShapeDtypeStruct((NPAD, 4), F32),
            jax.ShapeDtypeStruct((NPAD, HID), F32),
            jax.ShapeDtypeStruct((NPAD, HID), F32),
        ],
    )(hs, c4, am, as_, Wn1a, Wn1b, bn1r, Wn2r, bn2r, Wa, Wb, be1r)


def _tc_out(hs, Wout, bout_r):
    return pl.pallas_call(
        _out_body,
        grid=(NPAD // C_N,),
        in_specs=[
            pl.BlockSpec((C_N, HID), lambda i: (i, 0)),
            _full(HID, DOUT),
            _full(1, DOUT),
        ],
        out_specs=pl.BlockSpec((C_N, DOUT), lambda i: (i, 0)),
        out_shape=jax.ShapeDtypeStruct((NPAD, DOUT), F32),
    )(hs, Wout, bout_r)


def kernel(h, coords, edge_index, edge_attr, Win, bin_, Wout, bout,
           We1, be1, We2, be2, Wn1, bn1, Wn2, bn2, Wc1, bc1, Wc2):
    row = edge_index[0].astype(jnp.int32)
    col = edge_index[1].astype(jnp.int32)
    pad_e = EPAD - E
    col_flat = jnp.concatenate([col, jnp.full((pad_e,), NPAD - 1, jnp.int32)])
    row_flat = jnp.concatenate([row, jnp.zeros((pad_e,), jnp.int32)])
    col_pad = col_flat.reshape(NCHUNK, 1, GCH)
    row_pad = row_flat.reshape(NCHUNK, 1, GCH)
    ea_pad = jnp.concatenate(
        [edge_attr, jnp.zeros((pad_e, ED), F32)], axis=0)
    h_pad = jnp.concatenate([h, jnp.zeros((NPAD - N, DIN), F32)], axis=0)
    c4 = jnp.concatenate(
        [jnp.concatenate([coords, jnp.zeros((N, 1), F32)], axis=1),
         jnp.zeros((NPAD - N, 4), F32)], axis=0)

    zeros128 = jnp.zeros((NPAD, HID), F32)

    hs, ta, tb = _tc_embed(h_pad, Win, bin_.reshape(1, HID),
                           We1[0, 0:HID], We1[0, HID:2 * HID],
                           be1[0].reshape(1, HID))

    for l in range(L):
        wd = We1[l, 2 * HID:2 * HID + 1]
        We = We1[l, 2 * HID + 1:]
        aux = _sc_coords(c4.reshape(NPAD * 4), col_flat, row_flat)
        gc, gr = _sc_gather(ta, tb, col_pad, row_pad)
        msg_m, msg_s = _tc_edge(gc, gr, aux, ea_pad, wd, We, We2[l],
                                be2[l].reshape(1, HID), Wc1[l],
                                bc1[l].reshape(1, HID), Wc2[l])
        agg = _sc_scatter(msg_m, msg_s, col_pad, zeros128)
        if l < L - 1:
            hs, c4, ta, tb = _tc_node_proj(
                hs, c4, agg[0], agg[1],
                Wn1[l, :HID], Wn1[l, HID:], bn1[l].reshape(1, HID),
                Wn2[l], bn2[l].reshape(1, HID),
                We1[l + 1, 0:HID], We1[l + 1, HID:2 * HID],
                be1[l + 1].reshape(1, HID))
        else:
            hs, c4 = _tc_node(hs, c4, agg[0], agg[1],
                              Wn1[l, :HID], Wn1[l, HID:],
                              bn1[l].reshape(1, HID),
                              Wn2[l], bn2[l].reshape(1, HID))

    h_out = _tc_out(hs, Wout, bout.reshape(1, DOUT))
    return (h_out[:N], c4[:N, 0:3])
